# baseline (speedup 1.0000x reference)
import functools

import jax
import jax.numpy as jnp
from jax import lax
from jax.experimental import pallas as pl
from jax.experimental.pallas import tpu as pltpu
from jax.experimental.pallas import tpu_sc as plsc

_NUM_CORES = 2
_NUM_SUBCORES = 16
_NW = _NUM_CORES * _NUM_SUBCORES


def _iota16():
    return lax.iota(jnp.int32, 16)


def _full16(x):
    return jnp.zeros((16,), jnp.int32) + x


@functools.lru_cache(maxsize=None)
def _make_linearize(K: int, D: int):
    assert D == 32
    ncols = K // 128
    rem = K - ncols * 128
    cpw = (ncols + _NW - 1) // _NW
    nsteps = (cpw + 1) // 2
    mesh = plsc.VectorSubcoreMesh(core_axis_name="c", subcore_axis_name="s")

    @functools.partial(
        pl.kernel,
        out_type=jax.ShapeDtypeStruct((K * D // 128, 128), jnp.float32),
        mesh=mesh,
        scratch_types=[
            pltpu.VMEM((2, D, 128), jnp.float32),
            pltpu.VMEM((2, D, 128), jnp.float32),
            pltpu.VMEM((D, 64), jnp.float32),
            pltpu.SemaphoreType.DMA,
            pltpu.SemaphoreType.DMA,
        ],
        compiler_params=pltpu.CompilerParams(use_tc_tiling_on_sc=True, needs_layout_passes=False),
    )
    def linearize_kernel(tab_hbm, lin_hbm, g_v, o_v, gp_v, sem_r, sem_w):
        wid = lax.axis_index("s") * _NUM_CORES + lax.axis_index("c")
        lo = wid * cpw
        n = jnp.minimum(lo + cpw, ncols) - lo

        def read_start(c, b):
            pltpu.async_copy(
                tab_hbm.at[:, pl.ds(c * 128, 128)], g_v.at[b], sem_r)

        def read_wait(b):
            pltpu.make_async_copy(
                tab_hbm.at[:, pl.ds(0, 128)], g_v.at[b], sem_r).wait()

        def write_start(c, b):
            pltpu.async_copy(
                o_v.at[b], lin_hbm.at[pl.ds(c * (D * 128 // 128), D), :], sem_w)

        def write_wait(b):
            pltpu.make_async_copy(
                o_v.at[b], lin_hbm.at[pl.ds(0, D), :], sem_w).wait()

        def transpose(src, dst, width):
            nrows = width * D // 128

            @pl.loop(0, nrows)
            def _rows(r):
                for jg in range(8):
                    rows16 = _iota16() + 16 * (jg % 2)
                    cols16 = _full16(4 * r + (jg // 2))
                    v = plsc.load_gather(src, [rows16, cols16])
                    dst[r, pl.ds(16 * jg, 16)] = v

        @pl.when(n > 0)
        def _():
            read_start(lo, 0)

        @pl.loop(0, nsteps)
        def _outer(s):
            for b in range(2):
                i = 2 * s + b
                c = lo + i

                @pl.when(i < n)
                def _():
                    read_wait(b)

                    @pl.when(i + 1 < n)
                    def _():
                        read_start(c + 1, 1 - b)

                    @pl.when(i >= 2)
                    def _():
                        write_wait(b)

                    transpose(g_v.at[b], o_v.at[b], 128)
                    write_start(c, b)

        @pl.when(n >= 1)
        def _():
            write_wait(0)

        @pl.when(n >= 2)
        def _():
            write_wait(0)

        if rem:
            @pl.when(wid == _NW - 1)
            def _():
                pltpu.sync_copy(tab_hbm.at[:, pl.ds(ncols * 128, rem)], gp_v)
                nrows = rem * D // 128

                @pl.loop(0, nrows)
                def _rows(r):
                    for jg in range(8):
                        rows16 = _iota16() + 16 * (jg % 2)
                        cols16 = _full16(4 * r + (jg // 2))
                        v = plsc.load_gather(gp_v, [rows16, cols16])
                        o_v[0, r, pl.ds(16 * jg, 16)] = v

                pltpu.sync_copy(
                    o_v.at[0, pl.ds(0, nrows)],
                    lin_hbm.at[pl.ds(ncols * D, nrows), :])

    return linearize_kernel


@functools.lru_cache(maxsize=None)
def _make_gather_native(hist: int, batch: int, K: int, D: int):
    assert D == 32 and batch % 128 == 0 and hist % 2 == 0
    nchunks = hist // 2
    mesh = plsc.VectorSubcoreMesh(core_axis_name="c", subcore_axis_name="s")

    @functools.partial(
        pl.kernel,
        out_type=jax.ShapeDtypeStruct((hist, D, batch), jnp.float32),
        mesh=mesh,
        scratch_types=[
            pltpu.VMEM((hist, 128), jnp.int32),
            pltpu.VMEM((256,), jnp.int32),
            pltpu.VMEM((256,), jnp.int32),
            pltpu.VMEM((2, 256, 128), jnp.float32),
            pltpu.VMEM((2, 2, D, 128), jnp.float32),
            pltpu.SemaphoreType.DMA,
            pltpu.SemaphoreType.DMA,
        ],
        compiler_params=pltpu.CompilerParams(use_tc_tiling_on_sc=True, needs_layout_passes=False),
    )
    def gather_kernel(idx_hbm, lin_hbm, out_hbm, idx_all, rows0_v, rows1_v,
                      g4_v, t_v, sem_g, sem_o):
        rows_bufs = (rows0_v, rows1_v)
        wid = lax.axis_index("s") * _NUM_CORES + lax.axis_index("c")
        lane0 = wid * 128

        pltpu.sync_copy(idx_hbm.at[:, pl.ds(lane0, 128)], idx_all)

        def compute_rows(q, b):
            for hh in range(2):
                for g in range(8):
                    v = idx_all[2 * q + hh, pl.ds(16 * g, 16)]
                    rows_bufs[b][pl.ds(128 * hh + 16 * g, 16)] = (
                        lax.shift_right_logical(v, 2))

        def gather_start(b):
            pltpu.async_copy(lin_hbm.at[rows_bufs[b]], g4_v.at[b], sem_g)

        def gather_wait(b):
            pltpu.make_async_copy(
                lin_hbm.at[rows_bufs[b]], g4_v.at[b], sem_g).wait()

        def extract(q, b):
            for hh in range(2):
                for g in range(8):
                    iv = idx_all[2 * q + hh, pl.ds(16 * g, 16)]
                    colbase = lax.mul(jnp.bitwise_and(iv, 3), _full16(D))
                    rows16 = _iota16() + (128 * hh + 16 * g)

                    @pl.loop(0, D)
                    def _d(d):
                        v = plsc.load_gather(
                            g4_v.at[b], [rows16, colbase + d])
                        t_v[b, hh, d, pl.ds(16 * g, 16)] = v

        def write_start(q, b):
            for hh in range(2):
                pltpu.async_copy(
                    t_v.at[b, hh],
                    out_hbm.at[2 * q + hh, :, pl.ds(lane0, 128)], sem_o)

        def write_wait():
            pltpu.make_async_copy(
                t_v.at[0, 0], out_hbm.at[0, :, pl.ds(lane0, 128)], sem_o).wait()

        compute_rows(0, 0)
        gather_start(0)

        @pl.loop(0, nchunks // 2)
        def _outer(s):
            for b in range(2):
                q = 2 * s + b
                gather_wait(b)

                @pl.when(q + 1 < nchunks)
                def _():
                    compute_rows(q + 1, 1 - b)
                    gather_start(1 - b)

                @pl.when(q >= 2)
                def _():
                    write_wait()
                    write_wait()

                extract(q, b)
                write_start(q, b)

        write_wait()
        write_wait()
        write_wait()
        write_wait()

    return gather_kernel


def kernel(indices, table):
    batch, hist = indices.shape
    num_codes, dim = table.shape
    lin = _make_linearize(num_codes, dim)(table.T)
    out_t = _make_gather_native(hist, batch, num_codes, dim)(indices.T, lin)
    return out_t.transpose(2, 0, 1)


# speedup vs baseline: 1.8178x; 1.8178x over previous
import functools

import jax
import jax.numpy as jnp
from jax import lax
from jax.experimental import pallas as pl
from jax.experimental.pallas import tpu as pltpu
from jax.experimental.pallas import tpu_sc as plsc

_NUM_CORES = 2
_NUM_SUBCORES = 16
_NW = _NUM_CORES * _NUM_SUBCORES


def _iota16():
    return lax.iota(jnp.int32, 16)


def _full16(x):
    return jnp.zeros((16,), jnp.int32) + x


@functools.lru_cache(maxsize=None)
def _make_linearize(K: int, D: int):
    assert D == 32
    ncols = K // 128
    rem = K - ncols * 128
    cpw = (ncols + _NW - 1) // _NW
    nsteps = (cpw + 1) // 2
    mesh = plsc.VectorSubcoreMesh(core_axis_name="c", subcore_axis_name="s")

    @functools.partial(
        pl.kernel,
        out_type=jax.ShapeDtypeStruct((K * D // 128, 128), jnp.float32),
        mesh=mesh,
        scratch_types=[
            pltpu.VMEM((2, D, 128), jnp.float32),
            pltpu.VMEM((2, D, 128), jnp.float32),
            pltpu.VMEM((D, 64), jnp.float32),
            pltpu.SemaphoreType.DMA,
            pltpu.SemaphoreType.DMA,
        ],
        compiler_params=pltpu.CompilerParams(use_tc_tiling_on_sc=True, needs_layout_passes=False),
    )
    def linearize_kernel(tab_hbm, lin_hbm, g_v, o_v, gp_v, sem_r, sem_w):
        wid = lax.axis_index("s") * _NUM_CORES + lax.axis_index("c")
        lo = wid * cpw
        n = jnp.minimum(lo + cpw, ncols) - lo

        def read_start(c, b):
            pltpu.async_copy(
                tab_hbm.at[:, pl.ds(c * 128, 128)], g_v.at[b], sem_r)

        def read_wait(b):
            pltpu.make_async_copy(
                tab_hbm.at[:, pl.ds(0, 128)], g_v.at[b], sem_r).wait()

        def write_start(c, b):
            pltpu.async_copy(
                o_v.at[b], lin_hbm.at[pl.ds(c * (D * 128 // 128), D), :], sem_w)

        def write_wait(b):
            pltpu.make_async_copy(
                o_v.at[b], lin_hbm.at[pl.ds(0, D), :], sem_w).wait()

        def transpose(src, dst, width):
            nrows = width * D // 128

            @plsc.parallel_loop(0, nrows, unroll=4)
            def _rows(r):
                for jg in range(8):
                    rows16 = _iota16() + 16 * (jg % 2)
                    cols16 = _full16(4 * r + (jg // 2))
                    v = plsc.load_gather(src, [rows16, cols16])
                    dst[r, pl.ds(16 * jg, 16)] = v

        @pl.when(n > 0)
        def _():
            read_start(lo, 0)

        @pl.loop(0, nsteps)
        def _outer(s):
            for b in range(2):
                i = 2 * s + b
                c = lo + i

                @pl.when(i < n)
                def _():
                    read_wait(b)

                    @pl.when(i + 1 < n)
                    def _():
                        read_start(c + 1, 1 - b)

                    @pl.when(i >= 2)
                    def _():
                        write_wait(b)

                    transpose(g_v.at[b], o_v.at[b], 128)
                    write_start(c, b)

        @pl.when(n >= 1)
        def _():
            write_wait(0)

        @pl.when(n >= 2)
        def _():
            write_wait(0)

        if rem:
            @pl.when(wid == _NW - 1)
            def _():
                pltpu.sync_copy(tab_hbm.at[:, pl.ds(ncols * 128, rem)], gp_v)
                nrows = rem * D // 128

                @plsc.parallel_loop(0, nrows, unroll=4)
                def _rows(r):
                    for jg in range(8):
                        rows16 = _iota16() + 16 * (jg % 2)
                        cols16 = _full16(4 * r + (jg // 2))
                        v = plsc.load_gather(gp_v, [rows16, cols16])
                        o_v[0, r, pl.ds(16 * jg, 16)] = v

                pltpu.sync_copy(
                    o_v.at[0, pl.ds(0, nrows)],
                    lin_hbm.at[pl.ds(ncols * D, nrows), :])

    return linearize_kernel


@functools.lru_cache(maxsize=None)
def _make_gather_native(hist: int, batch: int, K: int, D: int):
    assert D == 32 and batch % 128 == 0 and hist % 2 == 0
    nchunks = hist // 2
    mesh = plsc.VectorSubcoreMesh(core_axis_name="c", subcore_axis_name="s")

    @functools.partial(
        pl.kernel,
        out_type=jax.ShapeDtypeStruct((hist, D, batch), jnp.float32),
        mesh=mesh,
        scratch_types=[
            pltpu.VMEM((hist, 128), jnp.int32),
            pltpu.VMEM((256,), jnp.int32),
            pltpu.VMEM((256,), jnp.int32),
            pltpu.VMEM((2, 256, 128), jnp.float32),
            pltpu.VMEM((2, 2, D, 128), jnp.float32),
            pltpu.SemaphoreType.DMA,
            pltpu.SemaphoreType.DMA,
        ],
        compiler_params=pltpu.CompilerParams(use_tc_tiling_on_sc=True, needs_layout_passes=False),
    )
    def gather_kernel(idx_hbm, lin_hbm, out_hbm, idx_all, rows0_v, rows1_v,
                      g4_v, t_v, sem_g, sem_o):
        rows_bufs = (rows0_v, rows1_v)
        wid = lax.axis_index("s") * _NUM_CORES + lax.axis_index("c")
        lane0 = wid * 128

        pltpu.sync_copy(idx_hbm.at[:, pl.ds(lane0, 128)], idx_all)

        def compute_rows(q, b):
            for hh in range(2):
                for g in range(8):
                    v = idx_all[2 * q + hh, pl.ds(16 * g, 16)]
                    rows_bufs[b][pl.ds(128 * hh + 16 * g, 16)] = (
                        lax.shift_right_logical(v, 2))

        def gather_start(b):
            pltpu.async_copy(lin_hbm.at[rows_bufs[b]], g4_v.at[b], sem_g)

        def gather_wait(b):
            pltpu.make_async_copy(
                lin_hbm.at[rows_bufs[b]], g4_v.at[b], sem_g).wait()

        def extract(q, b):
            for hh in range(2):
                for g in range(8):
                    iv = idx_all[2 * q + hh, pl.ds(16 * g, 16)]
                    colbase = lax.mul(jnp.bitwise_and(iv, 3), _full16(D))
                    rows16 = _iota16() + (128 * hh + 16 * g)

                    @plsc.parallel_loop(0, D, unroll=4)
                    def _d(d):
                        v = plsc.load_gather(
                            g4_v.at[b], [rows16, colbase + d])
                        t_v[b, hh, d, pl.ds(16 * g, 16)] = v

        def write_start(q, b):
            for hh in range(2):
                pltpu.async_copy(
                    t_v.at[b, hh],
                    out_hbm.at[2 * q + hh, :, pl.ds(lane0, 128)], sem_o)

        def write_wait():
            pltpu.make_async_copy(
                t_v.at[0, 0], out_hbm.at[0, :, pl.ds(lane0, 128)], sem_o).wait()

        compute_rows(0, 0)
        gather_start(0)

        @pl.loop(0, nchunks // 2)
        def _outer(s):
            for b in range(2):
                q = 2 * s + b
                gather_wait(b)

                @pl.when(q + 1 < nchunks)
                def _():
                    compute_rows(q + 1, 1 - b)
                    gather_start(1 - b)

                @pl.when(q >= 2)
                def _():
                    write_wait()
                    write_wait()

                extract(q, b)
                write_start(q, b)

        write_wait()
        write_wait()
        write_wait()
        write_wait()

    return gather_kernel


def kernel(indices, table):
    batch, hist = indices.shape
    num_codes, dim = table.shape
    lin = _make_linearize(num_codes, dim)(table.T)
    out_t = _make_gather_native(hist, batch, num_codes, dim)(indices.T, lin)
    return out_t.transpose(2, 0, 1)
